# baseline (speedup 1.0000x reference)
import functools

import jax
import jax.numpy as jnp
from jax import lax
from jax.experimental import pallas as pl
from jax.experimental.pallas import tpu as pltpu
from jax.experimental.pallas import tpu_sc as plsc

N_NODES = 10000
D_FEAT = 128
N_PAIRS = 320000

_info = plsc.get_sparse_core_info()
_NC, _NS, _L = _info.num_cores, _info.num_subcores, _info.num_lanes
_NW = _NC * _NS
_BPW = N_PAIRS // _NW
_CHUNK = 10240


_SPAD = 10240
_MV_BLK = _SPAD // 8


def _matvec_body(w_ref, b_ref, h_ref, o_ref):
    w2 = w_ref[0:128, :] + w_ref[128:256, :]
    s = jax.lax.dot_general(
        w2,
        h_ref[...],
        dimension_numbers=(((0,), (1,)), ((), ())),
        preferred_element_type=jnp.float32,
        precision=jax.lax.Precision.HIGHEST,
    )
    o_ref[...] = s + b_ref[0]


_matvec = pl.pallas_call(
    _matvec_body,
    grid=(8,),
    in_specs=[
        pl.BlockSpec((257, 1), lambda i: (0, 0)),
        pl.BlockSpec((1,), lambda i: (0,)),
        pl.BlockSpec((_MV_BLK, 128), lambda i: (i, 0)),
    ],
    out_specs=pl.BlockSpec((1, _MV_BLK), lambda i: (0, i)),
    out_shape=jax.ShapeDtypeStruct((1, _SPAD), jnp.float32),
)


_mesh = plsc.VectorSubcoreMesh(core_axis_name="c", subcore_axis_name="s")


@functools.partial(
    pl.kernel,
    mesh=_mesh,
    out_type=jax.ShapeDtypeStruct((N_PAIRS,), jnp.float32),
    compiler_params=pltpu.CompilerParams(needs_layout_passes=False),
    scratch_types=[
        pltpu.VMEM((_SPAD,), jnp.float32),
        pltpu.VMEM((2, _CHUNK), jnp.int32),
        pltpu.VMEM((_BPW,), jnp.float32),
    ],
)
def _sc_pool(s_hbm, idx_hbm, out_hbm, s_v, i01_v, o_v):
    wid = lax.axis_index("s") * _NC + lax.axis_index("c")
    base = wid * _BPW
    base_al = pl.multiple_of(
        jnp.minimum((base // 128) * 128, N_PAIRS - _CHUNK), 128
    )
    extra = base - base_al
    pltpu.sync_copy(s_hbm.at[0], s_v)
    pltpu.sync_copy(idx_hbm.at[:, pl.ds(base_al, _CHUNK)], i01_v)

    @plsc.parallel_loop(0, _BPW, step=_L, unroll=8)
    def _(off):
        v0 = plsc.load_gather(s_v, [i01_v[0, pl.ds(extra + off, _L)]])
        v1 = plsc.load_gather(s_v, [i01_v[1, pl.ds(extra + off, _L)]])
        o_v[pl.ds(off, _L)] = v0 + v1
    pltpu.sync_copy(o_v, out_hbm.at[pl.ds(base, _BPW)])


def kernel(h, forward_indices, W, b):
    s = _matvec(W, b, h)
    return _sc_pool(s, forward_indices).reshape(N_PAIRS, 1)


# speedup vs baseline: 1.0301x; 1.0301x over previous
import functools

import jax
import jax.numpy as jnp
from jax import lax
from jax.experimental import pallas as pl
from jax.experimental.pallas import tpu as pltpu
from jax.experimental.pallas import tpu_sc as plsc

N_NODES = 10000
D_FEAT = 128
N_PAIRS = 320000

_info = plsc.get_sparse_core_info()
_NC, _NS, _L = _info.num_cores, _info.num_subcores, _info.num_lanes
_NW = _NC * _NS
_BPW = N_PAIRS // _NW
_CHUNK = 10240


_SPAD = 10240
_MV_BLK = _SPAD // 8


def _matvec_body(w_ref, b_ref, h_ref, o_ref):
    w2 = w_ref[0:128, :] + w_ref[128:256, :]
    s = jax.lax.dot_general(
        w2,
        h_ref[...],
        dimension_numbers=(((0,), (1,)), ((), ())),
        preferred_element_type=jnp.float32,
        precision=jax.lax.Precision.HIGHEST,
    )
    o_ref[...] = s + b_ref[0]


_matvec = pl.pallas_call(
    _matvec_body,
    out_shape=jax.ShapeDtypeStruct((1, N_NODES), jnp.float32),
)


_mesh = plsc.VectorSubcoreMesh(core_axis_name="c", subcore_axis_name="s")


@functools.partial(
    pl.kernel,
    mesh=_mesh,
    out_type=jax.ShapeDtypeStruct((N_PAIRS,), jnp.float32),
    compiler_params=pltpu.CompilerParams(needs_layout_passes=False),
    scratch_types=[
        pltpu.VMEM((N_NODES,), jnp.float32),
        pltpu.VMEM((2, _CHUNK), jnp.int32),
        pltpu.VMEM((_BPW,), jnp.float32),
    ],
)
def _sc_pool(s_hbm, idx_hbm, out_hbm, s_v, i01_v, o_v):
    wid = lax.axis_index("s") * _NC + lax.axis_index("c")
    base = wid * _BPW
    base_al = pl.multiple_of(
        jnp.minimum((base // 128) * 128, N_PAIRS - _CHUNK), 128
    )
    extra = base - base_al
    pltpu.sync_copy(s_hbm.at[0], s_v)
    pltpu.sync_copy(idx_hbm.at[:, pl.ds(base_al, _CHUNK)], i01_v)

    @plsc.parallel_loop(0, _BPW, step=_L, unroll=8)
    def _(off):
        v0 = plsc.load_gather(s_v, [i01_v[0, pl.ds(extra + off, _L)]])
        v1 = plsc.load_gather(s_v, [i01_v[1, pl.ds(extra + off, _L)]])
        o_v[pl.ds(off, _L)] = v0 + v1
    pltpu.sync_copy(o_v, out_hbm.at[pl.ds(base, _BPW)])


def kernel(h, forward_indices, W, b):
    s = _matvec(W, b, h)
    return _sc_pool(s, forward_indices).reshape(N_PAIRS, 1)


# speedup vs baseline: 1.0344x; 1.0042x over previous
import functools

import jax
import jax.numpy as jnp
from jax import lax
from jax.experimental import pallas as pl
from jax.experimental.pallas import tpu as pltpu
from jax.experimental.pallas import tpu_sc as plsc

N_NODES = 10000
D_FEAT = 128
N_PAIRS = 320000

_info = plsc.get_sparse_core_info()
_NC, _NS, _L = _info.num_cores, _info.num_subcores, _info.num_lanes
_NW = _NC * _NS
_BPW = N_PAIRS // _NW
_CHUNK = 10240


_SPAD = 10240
_MV_BLK = _SPAD // 8


def _matvec_body(w_ref, b_ref, h_ref, o_ref):
    w2 = w_ref[0:128, :] + w_ref[128:256, :]
    s = jax.lax.dot_general(
        w2,
        h_ref[...],
        dimension_numbers=(((0,), (1,)), ((), ())),
        preferred_element_type=jnp.float32,
        precision=jax.lax.Precision.HIGHEST,
    )
    o_ref[...] = s + b_ref[0]


_matvec = pl.pallas_call(
    _matvec_body,
    out_shape=jax.ShapeDtypeStruct((1, N_NODES), jnp.float32),
)


_mesh = plsc.VectorSubcoreMesh(core_axis_name="c", subcore_axis_name="s")


_QA = 10240
_ROWS_A = _QA // 128
_QB = N_PAIRS - (_NW - 1) * _QA
_ROWS_B = _QB // 128
_OUT_ROWS = N_PAIRS // 128


@functools.partial(
    pl.kernel,
    mesh=_mesh,
    out_type=jax.ShapeDtypeStruct((_OUT_ROWS, 128), jnp.float32),
    compiler_params=pltpu.CompilerParams(needs_layout_passes=False),
    scratch_types=[
        pltpu.VMEM((N_NODES,), jnp.float32),
        pltpu.VMEM((2, _QA), jnp.int32),
        pltpu.VMEM((_ROWS_A, 128), jnp.float32),
    ],
)
def _sc_pool(s_hbm, idx_hbm, out_hbm, s_v, i01_v, o_v):
    wid = lax.axis_index("s") * _NC + lax.axis_index("c")
    pltpu.sync_copy(s_hbm.at[0], s_v)

    def run(nrows, pair_base, row_base):
        pltpu.sync_copy(
            idx_hbm.at[:, pl.ds(pair_base, nrows * 128)],
            i01_v.at[:, pl.ds(0, nrows * 128)],
        )

        @plsc.parallel_loop(0, nrows, step=1, unroll=2)
        def _(r):
            for cb in range(128 // _L):
                off = r * 128 + cb * _L
                v0 = plsc.load_gather(s_v, [i01_v[0, pl.ds(off, _L)]])
                v1 = plsc.load_gather(s_v, [i01_v[1, pl.ds(off, _L)]])
                o_v[r, pl.ds(cb * _L, _L)] = v0 + v1

        pltpu.sync_copy(
            o_v.at[pl.ds(0, nrows), :],
            out_hbm.at[pl.ds(row_base, nrows), :],
        )

    @pl.when(wid < _NW - 1)
    def _():
        run(_ROWS_A, wid * _QA, wid * _ROWS_A)

    @pl.when(wid == _NW - 1)
    def _():
        run(_ROWS_B, (_NW - 1) * _QA, (_NW - 1) * _ROWS_A)


def kernel(h, forward_indices, W, b):
    s = _matvec(W, b, h)
    return _sc_pool(s, forward_indices).reshape(N_PAIRS, 1)


# speedup vs baseline: 1.0651x; 1.0297x over previous
import functools

import jax
import jax.numpy as jnp
from jax import lax
from jax.experimental import pallas as pl
from jax.experimental.pallas import tpu as pltpu
from jax.experimental.pallas import tpu_sc as plsc

N_NODES = 10000
D_FEAT = 128
N_PAIRS = 320000

_info = plsc.get_sparse_core_info()
_NC, _NS, _L = _info.num_cores, _info.num_subcores, _info.num_lanes
_NW = _NC * _NS
_BPW = N_PAIRS // _NW
_CHUNK = 10240


_SPAD = 10240
_MV_BLK = _SPAD // 8


def _matvec_body(w_ref, b_ref, h_ref, o_ref):
    w2 = w_ref[0:128, :] + w_ref[128:256, :]
    s = jax.lax.dot_general(
        w2,
        h_ref[...],
        dimension_numbers=(((0,), (1,)), ((), ())),
        preferred_element_type=jnp.float32,
        precision=jax.lax.Precision.HIGHEST,
    )
    o_ref[...] = s + b_ref[0]


_matvec = pl.pallas_call(
    _matvec_body,
    out_shape=jax.ShapeDtypeStruct((1, N_NODES), jnp.float32),
)


_mesh = plsc.VectorSubcoreMesh(core_axis_name="c", subcore_axis_name="s")


_QA = 10240
_ROWS_A = _QA // 128
_QB = N_PAIRS - (_NW - 1) * _QA
_ROWS_B = _QB // 128
_OUT_ROWS = N_PAIRS // 128


@functools.partial(
    pl.kernel,
    mesh=_mesh,
    out_type=jax.ShapeDtypeStruct((_OUT_ROWS, 128), jnp.float32),
    compiler_params=pltpu.CompilerParams(needs_layout_passes=False),
    scratch_types=[
        pltpu.VMEM((N_NODES,), jnp.float32),
        pltpu.VMEM((2, _QA), jnp.int32),
        pltpu.VMEM((_ROWS_A, 128), jnp.float32),
        pltpu.SemaphoreType.DMA,
    ],
)
def _sc_pool(s_hbm, idx_hbm, out_hbm, s_v, i01_v, o_v, s_sem):
    wid = lax.axis_index("s") * _NC + lax.axis_index("c")
    s_cp = pltpu.async_copy(s_hbm.at[0], s_v, s_sem)

    def run(nrows, pair_base, row_base):
        pltpu.sync_copy(
            idx_hbm.at[:, pl.ds(pair_base, nrows * 128)],
            i01_v.at[:, pl.ds(0, nrows * 128)],
        )
        s_cp.wait()

        @plsc.parallel_loop(0, nrows, step=1, unroll=2)
        def _(r):
            for cb in range(128 // _L):
                off = r * 128 + cb * _L
                v0 = plsc.load_gather(s_v, [i01_v[0, pl.ds(off, _L)]])
                v1 = plsc.load_gather(s_v, [i01_v[1, pl.ds(off, _L)]])
                o_v[r, pl.ds(cb * _L, _L)] = v0 + v1

        pltpu.sync_copy(
            o_v.at[pl.ds(0, nrows), :],
            out_hbm.at[pl.ds(row_base, nrows), :],
        )

    @pl.when(wid < _NW - 1)
    def _():
        run(_ROWS_A, wid * _QA, wid * _ROWS_A)

    @pl.when(wid == _NW - 1)
    def _():
        run(_ROWS_B, (_NW - 1) * _QA, (_NW - 1) * _ROWS_A)


def kernel(h, forward_indices, W, b):
    s = _matvec(W, b, h)
    return _sc_pool(s, forward_indices).reshape(N_PAIRS, 1)


# speedup vs baseline: 1.1677x; 1.0964x over previous
import functools

import jax
import jax.numpy as jnp
from jax import lax
from jax.experimental import pallas as pl
from jax.experimental.pallas import tpu as pltpu
from jax.experimental.pallas import tpu_sc as plsc

N_NODES = 10000
D_FEAT = 128
N_PAIRS = 320000

_info = plsc.get_sparse_core_info()
_NC, _NS, _L = _info.num_cores, _info.num_subcores, _info.num_lanes
_NW = _NC * _NS
_BPW = N_PAIRS // _NW
_CHUNK = 10240


_SPAD = 10240
_MV_BLK = _SPAD // 8


def _matvec_body(w_ref, b_ref, h_ref, o_ref):
    w2 = w_ref[0:128, :] + w_ref[128:256, :]
    s = jax.lax.dot_general(
        w2,
        h_ref[...],
        dimension_numbers=(((0,), (1,)), ((), ())),
        preferred_element_type=jnp.float32,
    )
    o_ref[...] = s + b_ref[0]


_matvec = pl.pallas_call(
    _matvec_body,
    out_shape=jax.ShapeDtypeStruct((1, N_NODES), jnp.float32),
)


_mesh = plsc.VectorSubcoreMesh(core_axis_name="c", subcore_axis_name="s")


_QA = 10240
_ROWS_A = _QA // 128
_QB = N_PAIRS - (_NW - 1) * _QA
_ROWS_B = _QB // 128
_OUT_ROWS = N_PAIRS // 128


@functools.partial(
    pl.kernel,
    mesh=_mesh,
    out_type=jax.ShapeDtypeStruct((_OUT_ROWS, 128), jnp.float32),
    compiler_params=pltpu.CompilerParams(needs_layout_passes=False),
    scratch_types=[
        pltpu.VMEM((N_NODES,), jnp.float32),
        pltpu.VMEM((2, _QA), jnp.int32),
        pltpu.VMEM((_ROWS_A, 128), jnp.float32),
        pltpu.SemaphoreType.DMA,
    ],
)
def _sc_pool(s_hbm, idx_hbm, out_hbm, s_v, i01_v, o_v, s_sem):
    wid = lax.axis_index("s") * _NC + lax.axis_index("c")
    s_cp = pltpu.async_copy(s_hbm.at[0], s_v, s_sem)

    def run(nrows, pair_base, row_base):
        pltpu.sync_copy(
            idx_hbm.at[:, pl.ds(pair_base, nrows * 128)],
            i01_v.at[:, pl.ds(0, nrows * 128)],
        )
        s_cp.wait()

        @plsc.parallel_loop(0, nrows, step=1, unroll=2)
        def _(r):
            for cb in range(128 // _L):
                off = r * 128 + cb * _L
                v0 = plsc.load_gather(s_v, [i01_v[0, pl.ds(off, _L)]])
                v1 = plsc.load_gather(s_v, [i01_v[1, pl.ds(off, _L)]])
                o_v[r, pl.ds(cb * _L, _L)] = v0 + v1

        pltpu.sync_copy(
            o_v.at[pl.ds(0, nrows), :],
            out_hbm.at[pl.ds(row_base, nrows), :],
        )

    @pl.when(wid < _NW - 1)
    def _():
        run(_ROWS_A, wid * _QA, wid * _ROWS_A)

    @pl.when(wid == _NW - 1)
    def _():
        run(_ROWS_B, (_NW - 1) * _QA, (_NW - 1) * _ROWS_A)


def kernel(h, forward_indices, W, b):
    s = _matvec(W, b, h)
    return _sc_pool(s, forward_indices).reshape(N_PAIRS, 1)
